# Initial kernel scaffold; baseline (speedup 1.0000x reference)
import jax
import jax.numpy as jnp
from jax.experimental import pallas as pl


def kernel(inputs, table):
    raise NotImplementedError("write your pallas kernel here")


# speedup vs baseline: 4.8500x; 4.8500x over previous
import functools

import jax
import jax.numpy as jnp
from jax import lax
from jax.experimental import pallas as pl
from jax.experimental.pallas import tpu as pltpu
from jax.experimental.pallas import tpu_sc as plsc

_VOCAB = 1000000
_EMBED = 32
_BATCH = 16384
_HIST = 200

_B = _BATCH * _HIST
_NC = 2
_NS = 16
_NW = _NC * _NS
_BPW = _B // _NW
_CHUNK = 2048
_NSTEPS = _BPW // _CHUNK


def _gather_body(idx_hbm, table_hbm, out_hbm, idx_v, rows_v, sem):
  wid = lax.axis_index("s") * _NC + lax.axis_index("c")
  base = wid * _BPW

  def step(i, carry):
    off = base + i * _CHUNK
    pltpu.sync_copy(idx_hbm.at[pl.ds(off, _CHUNK)], idx_v)
    pltpu.async_copy(table_hbm.at[idx_v], rows_v, sem).wait()
    pltpu.sync_copy(rows_v, out_hbm.at[pl.ds(off, _CHUNK)])
    return carry

  lax.fori_loop(0, _NSTEPS, step, 0)


_gather = functools.partial(
    pl.kernel,
    out_type=jax.ShapeDtypeStruct((_B, _EMBED), jnp.float32),
    mesh=plsc.VectorSubcoreMesh(core_axis_name="c", subcore_axis_name="s"),
    scratch_types=[
        pltpu.VMEM((_CHUNK,), jnp.int32),
        pltpu.VMEM((_CHUNK, _EMBED), jnp.float32),
        pltpu.SemaphoreType.DMA,
    ],
    compiler_params=pltpu.CompilerParams(use_tc_tiling_on_sc=False),
)(_gather_body)


def _mask_body(x_ref, o_ref):
  o_ref[...] = x_ref[...] != 0


_mask = pl.pallas_call(
    _mask_body,
    out_shape=jax.ShapeDtypeStruct((_BATCH, _HIST), jnp.bool_),
    grid=(16,),
    in_specs=[pl.BlockSpec((_BATCH // 16, _HIST), lambda i: (i, 0))],
    out_specs=pl.BlockSpec((_BATCH // 16, _HIST), lambda i: (i, 0)),
)


@jax.jit
def kernel(inputs, table):
  flat_idx = inputs.reshape(_B)
  rows = _gather(flat_idx, table)
  mask = _mask(inputs)
  return rows.reshape(_BATCH, _HIST, _EMBED), mask


# speedup vs baseline: 4.9079x; 1.0119x over previous
import functools

import jax
import jax.numpy as jnp
from jax import lax
from jax.experimental import pallas as pl
from jax.experimental.pallas import tpu as pltpu
from jax.experimental.pallas import tpu_sc as plsc

_VOCAB = 1000000
_EMBED = 32
_BATCH = 16384
_HIST = 200

_B = _BATCH * _HIST
_NC = 2
_NS = 16
_NW = _NC * _NS
_BPW = _B // _NW
_CHUNK = 1600
_NSTEPS = _BPW // _CHUNK


def _gather_body(idx_hbm, table_hbm, out_hbm, idx_v, rows_v, gsem, osem0,
                 osem1):
  wid = lax.axis_index("s") * _NC + lax.axis_index("c")
  base = wid * _BPW
  osems = (osem0, osem1)

  def do_step(i, b):
    off = base + i * _CHUNK
    pltpu.sync_copy(idx_hbm.at[pl.ds(off, _CHUNK)], idx_v.at[b])

    @pl.when(i >= 2)
    def _():
      pltpu.make_async_copy(
          rows_v.at[b], out_hbm.at[pl.ds(off, _CHUNK)], osems[b]).wait()

    pltpu.async_copy(table_hbm.at[idx_v.at[b]], rows_v.at[b], gsem).wait()
    pltpu.async_copy(rows_v.at[b], out_hbm.at[pl.ds(off, _CHUNK)], osems[b])

  def pair(g, carry):
    do_step(g * 2, 0)
    do_step(g * 2 + 1, 1)
    return carry

  lax.fori_loop(0, _NSTEPS // 2, pair, 0)
  pltpu.make_async_copy(
      rows_v.at[0], out_hbm.at[pl.ds(base, _CHUNK)], osem0).wait()
  pltpu.make_async_copy(
      rows_v.at[1], out_hbm.at[pl.ds(base, _CHUNK)], osem1).wait()


_gather = functools.partial(
    pl.kernel,
    out_type=jax.ShapeDtypeStruct((_B, _EMBED), jnp.float32),
    mesh=plsc.VectorSubcoreMesh(core_axis_name="c", subcore_axis_name="s"),
    scratch_types=[
        pltpu.VMEM((2, _CHUNK), jnp.int32),
        pltpu.VMEM((2, _CHUNK, _EMBED), jnp.float32),
        pltpu.SemaphoreType.DMA,
        pltpu.SemaphoreType.DMA,
        pltpu.SemaphoreType.DMA,
    ],
    compiler_params=pltpu.CompilerParams(use_tc_tiling_on_sc=False),
)(_gather_body)


def _mask_body(x_ref, o_ref):
  o_ref[...] = x_ref[...] != 0


_mask = pl.pallas_call(
    _mask_body,
    out_shape=jax.ShapeDtypeStruct((_BATCH, _HIST), jnp.bool_),
    grid=(16,),
    in_specs=[pl.BlockSpec((_BATCH // 16, _HIST), lambda i: (i, 0))],
    out_specs=pl.BlockSpec((_BATCH // 16, _HIST), lambda i: (i, 0)),
)


@jax.jit
def kernel(inputs, table):
  flat_idx = inputs.reshape(_B)
  rows = _gather(flat_idx, table)
  mask = _mask(inputs)
  return rows.reshape(_BATCH, _HIST, _EMBED), mask


# speedup vs baseline: 4.9098x; 1.0004x over previous
import functools

import jax
import jax.numpy as jnp
from jax import lax
from jax.experimental import pallas as pl
from jax.experimental.pallas import tpu as pltpu
from jax.experimental.pallas import tpu_sc as plsc

_VOCAB = 1000000
_EMBED = 32
_BATCH = 16384
_HIST = 200

_NC = 2
_NS = 16
_NW = _NC * _NS
_RPW = _BATCH // _NW
_RC = 8
_NSTEPS = _RPW // _RC


def _gather_body(idx_hbm, table_hbm, out_hbm, idx_v, rows_v, gsem, osem0,
                 osem1):
  wid = lax.axis_index("s") * _NC + lax.axis_index("c")
  base = wid * _RPW
  osems = (osem0, osem1)

  def do_step(i, b):
    row0 = base + i * _RC
    pltpu.sync_copy(idx_hbm.at[pl.ds(row0, _RC), :], idx_v.at[b])

    @pl.when(i >= 2)
    def _():
      pltpu.make_async_copy(
          rows_v.at[b], out_hbm.at[pl.ds(row0, _RC)], osems[b]).wait()

    for j in range(_RC):
      pltpu.async_copy(table_hbm.at[idx_v.at[b, j]], rows_v.at[b, j], gsem)
    for j in range(_RC):
      pltpu.make_async_copy(
          table_hbm.at[idx_v.at[b, j]], rows_v.at[b, j], gsem).wait()

    pltpu.async_copy(rows_v.at[b], out_hbm.at[pl.ds(row0, _RC)], osems[b])

  def pair(g, carry):
    do_step(g * 2, 0)
    do_step(g * 2 + 1, 1)
    return carry

  lax.fori_loop(0, _NSTEPS // 2, pair, 0)
  pltpu.make_async_copy(
      rows_v.at[0], out_hbm.at[pl.ds(base, _RC)], osem0).wait()
  pltpu.make_async_copy(
      rows_v.at[1], out_hbm.at[pl.ds(base, _RC)], osem1).wait()


_gather = functools.partial(
    pl.kernel,
    out_type=jax.ShapeDtypeStruct((_BATCH, _HIST, _EMBED), jnp.float32),
    mesh=plsc.VectorSubcoreMesh(core_axis_name="c", subcore_axis_name="s"),
    scratch_types=[
        pltpu.VMEM((2, _RC, _HIST), jnp.int32),
        pltpu.VMEM((2, _RC, _HIST, _EMBED), jnp.float32),
        pltpu.SemaphoreType.DMA,
        pltpu.SemaphoreType.DMA,
        pltpu.SemaphoreType.DMA,
    ],
    compiler_params=pltpu.CompilerParams(use_tc_tiling_on_sc=False),
)(_gather_body)


def _mask_body(x_ref, o_ref):
  o_ref[...] = x_ref[...] != 0


_mask = pl.pallas_call(
    _mask_body,
    out_shape=jax.ShapeDtypeStruct((_BATCH, _HIST), jnp.bool_),
    grid=(16,),
    in_specs=[pl.BlockSpec((_BATCH // 16, _HIST), lambda i: (i, 0))],
    out_specs=pl.BlockSpec((_BATCH // 16, _HIST), lambda i: (i, 0)),
)


@jax.jit
def kernel(inputs, table):
  rows = _gather(inputs, table)
  mask = _mask(inputs)
  return rows, mask
